# baseline (speedup 1.0000x reference)
import functools

import jax
import jax.numpy as jnp
from jax import lax
from jax.experimental import pallas as pl
from jax.experimental.pallas import tpu as pltpu

N = 1024
NE = 3
C = 2
E = 32768
P = 4096
DH = 128
DS = C * DH
RB = 256


def _deg_body(a_ref, cs_ref, f1t_ref, f2t_ref, y_ref, dinv_ref, z_ref,
              racc, vacc):
    rb = pl.program_id(0)
    nrb = pl.num_programs(0)

    @pl.when(rb == 0)
    def _():
        racc[...] = lax.dot_general(cs_ref[...], f1t_ref[...],
                                    (((0,), (0,)), ((), ())),
                                    preferred_element_type=jnp.float32)
        vacc[...] = jnp.zeros_like(vacc)

    rblk = racc[pl.ds(rb * RB, RB), :]
    acc = vacc[...]
    for k in range(NE):
        res = lax.dot_general(a_ref[k], rblk, (((0,), (0,)), ((), ())),
                              preferred_element_type=jnp.float32)
        acc = acc + res * f2t_ref[k:k + 1, :]
    vacc[...] = acc

    @pl.when(rb == nrb - 1)
    def _():
        deg = 1.0 + vacc[...]
        dinv = jnp.where(deg > 0.0, lax.rsqrt(deg), 0.0)
        dinv_ref[...] = dinv
        y = y_ref[...]
        z_ref[...] = jnp.concatenate([dinv[:, 0:1] * y, dinv[:, 1:2] * y],
                                     axis=1)


def _deg_pass(A, cs, f1t, f2t, Y):
    grid = (N // RB,)
    return pl.pallas_call(
        _deg_body,
        grid=grid,
        in_specs=[
            pl.BlockSpec((NE, RB, N), lambda rb: (0, rb, 0)),
            pl.BlockSpec((NE, N), lambda rb: (0, 0)),
            pl.BlockSpec((NE, C), lambda rb: (0, 0)),
            pl.BlockSpec((NE, C), lambda rb: (0, 0)),
            pl.BlockSpec((N, DH), lambda rb: (0, 0)),
        ],
        out_specs=[
            pl.BlockSpec((N, C), lambda rb: (0, 0)),
            pl.BlockSpec((N, DS), lambda rb: (0, 0)),
        ],
        out_shape=[
            jax.ShapeDtypeStruct((N, C), jnp.float32),
            jax.ShapeDtypeStruct((N, DS), jnp.float32),
        ],
        scratch_shapes=[
            pltpu.VMEM((N, C), jnp.float32),
            pltpu.VMEM((N, C), jnp.float32),
        ],
    )(A, cs, f1t, f2t, Y)


def _stage_body(a_ref, zin_ref, scale_ref, z_ref, dinv_ref, b_ref, out_ref,
                acc, *, epilogue):
    j = pl.program_id(1)

    @pl.when(j == 0)
    def _():
        acc[...] = jnp.zeros_like(acc)

    res = lax.dot_general(a_ref[0], zin_ref[...], (((0,), (0,)), ((), ())),
                          preferred_element_type=jnp.float32)
    acc[...] += res * scale_ref[0]

    @pl.when(j == NE - 1)
    def _():
        if epilogue:
            dinvb = dinv_ref[...]
            dcols = jnp.concatenate(
                [jnp.broadcast_to(dinvb[:, 0:1], (RB, DH)),
                 jnp.broadcast_to(dinvb[:, 1:2], (RB, DH))], axis=1)
            out_ref[...] = jnp.maximum(
                dcols * (acc[...] + z_ref[...]) + b_ref[...], 0.0)
        else:
            out_ref[...] = acc[...]


def _stage(A, Zin, scale, Z, dinv, bstack, epilogue):
    grid = (N // RB, NE)
    body = functools.partial(_stage_body, epilogue=epilogue)
    return pl.pallas_call(
        body,
        grid=grid,
        in_specs=[
            pl.BlockSpec((1, N, RB), lambda io, j: (j, 0, io)),
            pl.BlockSpec((N, DS), lambda io, j: (0, 0)),
            pl.BlockSpec((1, 1, DS), lambda io, j: (j, 0, 0)),
            pl.BlockSpec((RB, DS), lambda io, j: (io, 0)),
            pl.BlockSpec((RB, C), lambda io, j: (io, 0)),
            pl.BlockSpec((1, DS), lambda io, j: (0, 0)),
        ],
        out_specs=pl.BlockSpec((RB, DS), lambda io, j: (io, 0)),
        out_shape=jax.ShapeDtypeStruct((N, DS), jnp.float32),
        scratch_shapes=[pltpu.VMEM((RB, DS), jnp.float32)],
    )(A, Zin, scale, Z, dinv, bstack)


def _xw_body(x_ref, w_ref, out_ref):
    out_ref[...] = jnp.dot(x_ref[...], w_ref[...],
                           preferred_element_type=jnp.float32)


def _xw(X, W):
    return pl.pallas_call(
        _xw_body,
        out_shape=jax.ShapeDtypeStruct((N, DH), jnp.float32),
    )(X, W)


MB = 512


def _mlp_body(bu_ref, bv_ref, t_ref, m1a_ref, m1b_ref, b1_ref, m2_ref,
              b2_ref, m3_ref, b3_ref, bp_ref, loss_ref, lacc):
    i = pl.program_id(0)

    @pl.when(i == 0)
    def _():
        lacc[...] = jnp.zeros_like(lacc)

    h = jnp.dot(bu_ref[...], m1a_ref[...], preferred_element_type=jnp.float32)
    h += jnp.dot(bv_ref[...], m1b_ref[...], preferred_element_type=jnp.float32)
    h = jnp.maximum(h + b1_ref[...], 0.0)
    h = jnp.maximum(jnp.dot(h, m2_ref[...], preferred_element_type=jnp.float32)
                    + b2_ref[...], 0.0)
    logits = jnp.dot(h, m3_ref[...], preferred_element_type=jnp.float32) \
        + b3_ref[...]
    m = jnp.max(logits, axis=-1, keepdims=True)
    e = jnp.exp(logits - m)
    bp = e / jnp.sum(e, axis=-1, keepdims=True)
    bp_ref[...] = bp

    mm = jnp.max(bp, axis=-1, keepdims=True)
    lse = mm + jnp.log(jnp.sum(jnp.exp(bp - mm), axis=-1, keepdims=True))
    t = t_ref[...]
    bpt = bp[:, 0:1] * (1.0 - t) + bp[:, 1:2] * t
    lacc[...] += jnp.sum(lse - bpt, axis=0, keepdims=True)

    @pl.when(i == pl.num_programs(0) - 1)
    def _():
        loss_ref[...] = lacc[...] * (1.0 / P)


def _mlp(Bu, Bv, targetf, M1a, M1b, b1, M2, b2, M3, b3):
    grid = (P // MB,)
    return pl.pallas_call(
        _mlp_body,
        grid=grid,
        in_specs=[
            pl.BlockSpec((MB, DS), lambda i: (i, 0)),
            pl.BlockSpec((MB, DS), lambda i: (i, 0)),
            pl.BlockSpec((MB, 1), lambda i: (i, 0)),
            pl.BlockSpec((DS, DS), lambda i: (0, 0)),
            pl.BlockSpec((DS, DS), lambda i: (0, 0)),
            pl.BlockSpec((1, DS), lambda i: (0, 0)),
            pl.BlockSpec((DS, DS // 2), lambda i: (0, 0)),
            pl.BlockSpec((1, DS // 2), lambda i: (0, 0)),
            pl.BlockSpec((DS // 2, 2), lambda i: (0, 0)),
            pl.BlockSpec((1, 2), lambda i: (0, 0)),
        ],
        out_specs=[
            pl.BlockSpec((MB, 2), lambda i: (i, 0)),
            pl.BlockSpec((1, 1), lambda i: (0, 0)),
        ],
        out_shape=[
            jax.ShapeDtypeStruct((P, 2), jnp.float32),
            jax.ShapeDtypeStruct((1, 1), jnp.float32),
        ],
        scratch_shapes=[pltpu.VMEM((1, 1), jnp.float32)],
    )(Bu, Bv, targetf, M1a, M1b, b1, M2, b2, M3, b3)


def _build_adj(edge_index, edge_value):
    mats = []
    for j in range(NE):
        mats.append(jnp.zeros((N, N), jnp.float32)
                    .at[edge_index[j, 0], edge_index[j, 1]]
                    .add(edge_value[j]))
    A = jnp.stack(mats)
    return A, A.sum(axis=1)


def _side(edge_index, edge_value, X, Wg, bg, Wgt1, Wgt2):
    f1 = jax.nn.softmax(Wgt1, axis=1)
    f2 = jax.nn.softmax(Wgt2, axis=1)
    f1t = f1.T
    f2t = f2.T
    A, cs = _build_adj(edge_index, edge_value)
    Y = _xw(X, Wg)
    dinv, Z = _deg_pass(A, cs, f1t, f2t, Y)
    scale1 = jnp.repeat(f1t, DH, axis=1)[:, None, :]
    scale2 = jnp.repeat(f2t, DH, axis=1)[:, None, :]
    bstack = jnp.tile(bg, (2,))[None, :]
    T1 = _stage(A, Z, scale1, Z, dinv, bstack, epilogue=False)
    Xout = _stage(A, T1, scale2, Z, dinv, bstack, epilogue=True)
    return Xout, f1, f2


def kernel(edge_index_u, edge_value_u, X_u, edge_index_v, edge_value_v, X_v,
           index_list, Wgt1_u, Wgt2_u, Wgt1_v, Wgt2_v, Wg_u, bg_u, Wg_v, bg_v,
           M1, b1, M2, b2, M3, b3):
    Xu_, f1u, f2u = _side(edge_index_u, edge_value_u, X_u, Wg_u, bg_u,
                          Wgt1_u, Wgt2_u)
    Xv_, f1v, f2v = _side(edge_index_v, edge_value_v, X_v, Wg_v, bg_v,
                          Wgt1_v, Wgt2_v)

    u_idx = index_list[:, 0]
    v_idx = index_list[:, 1]
    target = index_list[:, 2]
    targetf = target.astype(jnp.float32)

    Bu = Xu_[u_idx]
    Bv = Xv_[v_idx]

    Bp, loss2 = _mlp(Bu, Bv, targetf[:, None], M1[:DS], M1[DS:], b1[None, :],
                     M2, b2[None, :], M3, b3[None, :])
    loss = loss2.reshape(())
    return (Xu_, Xv_, f1u, f2u, f1v, f2v, loss, Bp, targetf)


# speedup vs baseline: 3.1150x; 3.1150x over previous
import functools

import jax
import jax.numpy as jnp
from jax import lax
from jax.experimental import pallas as pl
from jax.experimental.pallas import tpu as pltpu
from jax.experimental.pallas import tpu_sc as plsc

N = 1024
NE = 3
C = 2
E = 32768
P = 4096
DH = 128
DS = C * DH
RB = 256


def _deg_body(a_ref, cs_ref, f1t_ref, f2t_ref, y_ref, dinv_ref, z_ref,
              racc, vacc):
    rb = pl.program_id(0)
    nrb = pl.num_programs(0)

    @pl.when(rb == 0)
    def _():
        racc[...] = lax.dot_general(cs_ref[...], f1t_ref[...],
                                    (((0,), (0,)), ((), ())),
                                    preferred_element_type=jnp.float32)
        vacc[...] = jnp.zeros_like(vacc)

    rblk = racc[pl.ds(rb * RB, RB), :]
    acc = vacc[...]
    for k in range(NE):
        res = lax.dot_general(a_ref[k], rblk, (((0,), (0,)), ((), ())),
                              preferred_element_type=jnp.float32)
        acc = acc + res * f2t_ref[k:k + 1, :]
    vacc[...] = acc

    @pl.when(rb == nrb - 1)
    def _():
        deg = 1.0 + vacc[...]
        dinv = jnp.where(deg > 0.0, lax.rsqrt(deg), 0.0)
        dinv_ref[...] = dinv
        y = y_ref[...]
        z_ref[...] = jnp.concatenate([dinv[:, 0:1] * y, dinv[:, 1:2] * y],
                                     axis=1)


def _deg_pass(A, cs, f1t, f2t, Y):
    grid = (N // RB,)
    return pl.pallas_call(
        _deg_body,
        grid=grid,
        in_specs=[
            pl.BlockSpec((NE, RB, N), lambda rb: (0, rb, 0)),
            pl.BlockSpec((NE, N), lambda rb: (0, 0)),
            pl.BlockSpec((NE, C), lambda rb: (0, 0)),
            pl.BlockSpec((NE, C), lambda rb: (0, 0)),
            pl.BlockSpec((N, DH), lambda rb: (0, 0)),
        ],
        out_specs=[
            pl.BlockSpec((N, C), lambda rb: (0, 0)),
            pl.BlockSpec((N, DS), lambda rb: (0, 0)),
        ],
        out_shape=[
            jax.ShapeDtypeStruct((N, C), jnp.float32),
            jax.ShapeDtypeStruct((N, DS), jnp.float32),
        ],
        scratch_shapes=[
            pltpu.VMEM((N, C), jnp.float32),
            pltpu.VMEM((N, C), jnp.float32),
        ],
    )(A, cs, f1t, f2t, Y)


def _stage_body(a_ref, zin_ref, scale_ref, z_ref, dinv_ref, b_ref, out_ref,
                acc, *, epilogue):
    j = pl.program_id(1)

    @pl.when(j == 0)
    def _():
        acc[...] = jnp.zeros_like(acc)

    res = lax.dot_general(a_ref[0], zin_ref[...], (((0,), (0,)), ((), ())),
                          preferred_element_type=jnp.float32)
    acc[...] += res * scale_ref[0]

    @pl.when(j == NE - 1)
    def _():
        if epilogue:
            dinvb = dinv_ref[...]
            dcols = jnp.concatenate(
                [jnp.broadcast_to(dinvb[:, 0:1], (RB, DH)),
                 jnp.broadcast_to(dinvb[:, 1:2], (RB, DH))], axis=1)
            out_ref[...] = jnp.maximum(
                dcols * (acc[...] + z_ref[...]) + b_ref[...], 0.0)
        else:
            out_ref[...] = acc[...]


def _stage(A, Zin, scale, Z, dinv, bstack, epilogue):
    grid = (N // RB, NE)
    body = functools.partial(_stage_body, epilogue=epilogue)
    return pl.pallas_call(
        body,
        grid=grid,
        in_specs=[
            pl.BlockSpec((1, N, RB), lambda io, j: (j, 0, io)),
            pl.BlockSpec((N, DS), lambda io, j: (0, 0)),
            pl.BlockSpec((1, 1, DS), lambda io, j: (j, 0, 0)),
            pl.BlockSpec((RB, DS), lambda io, j: (io, 0)),
            pl.BlockSpec((RB, C), lambda io, j: (io, 0)),
            pl.BlockSpec((1, DS), lambda io, j: (0, 0)),
        ],
        out_specs=pl.BlockSpec((RB, DS), lambda io, j: (io, 0)),
        out_shape=jax.ShapeDtypeStruct((N, DS), jnp.float32),
        scratch_shapes=[pltpu.VMEM((RB, DS), jnp.float32)],
    )(A, Zin, scale, Z, dinv, bstack)


def _xw_body(x_ref, w_ref, out_ref):
    out_ref[...] = jnp.dot(x_ref[...], w_ref[...],
                           preferred_element_type=jnp.float32)


def _xw(X, W):
    return pl.pallas_call(
        _xw_body,
        out_shape=jax.ShapeDtypeStruct((N, DH), jnp.float32),
    )(X, W)


MB = 512


def _mlp_body(bu_ref, bv_ref, t_ref, m1a_ref, m1b_ref, b1_ref, m2_ref,
              b2_ref, m3_ref, b3_ref, bp_ref, loss_ref, lacc):
    i = pl.program_id(0)

    @pl.when(i == 0)
    def _():
        lacc[...] = jnp.zeros_like(lacc)

    h = jnp.dot(bu_ref[...], m1a_ref[...], preferred_element_type=jnp.float32)
    h += jnp.dot(bv_ref[...], m1b_ref[...], preferred_element_type=jnp.float32)
    h = jnp.maximum(h + b1_ref[...], 0.0)
    h = jnp.maximum(jnp.dot(h, m2_ref[...], preferred_element_type=jnp.float32)
                    + b2_ref[...], 0.0)
    logits = jnp.dot(h, m3_ref[...], preferred_element_type=jnp.float32) \
        + b3_ref[...]
    m = jnp.max(logits, axis=-1, keepdims=True)
    e = jnp.exp(logits - m)
    bp = e / jnp.sum(e, axis=-1, keepdims=True)
    bp_ref[...] = bp

    mm = jnp.max(bp, axis=-1, keepdims=True)
    lse = mm + jnp.log(jnp.sum(jnp.exp(bp - mm), axis=-1, keepdims=True))
    t = t_ref[...]
    bpt = bp[:, 0:1] * (1.0 - t) + bp[:, 1:2] * t
    lacc[...] += jnp.sum(lse - bpt, axis=0, keepdims=True)

    @pl.when(i == pl.num_programs(0) - 1)
    def _():
        loss_ref[...] = lacc[...] * (1.0 / P)


def _mlp(Bu, Bv, targetf, M1a, M1b, b1, M2, b2, M3, b3):
    grid = (P // MB,)
    return pl.pallas_call(
        _mlp_body,
        grid=grid,
        in_specs=[
            pl.BlockSpec((MB, DS), lambda i: (i, 0)),
            pl.BlockSpec((MB, DS), lambda i: (i, 0)),
            pl.BlockSpec((MB, 1), lambda i: (i, 0)),
            pl.BlockSpec((DS, DS), lambda i: (0, 0)),
            pl.BlockSpec((DS, DS), lambda i: (0, 0)),
            pl.BlockSpec((1, DS), lambda i: (0, 0)),
            pl.BlockSpec((DS, DS // 2), lambda i: (0, 0)),
            pl.BlockSpec((1, DS // 2), lambda i: (0, 0)),
            pl.BlockSpec((DS // 2, 2), lambda i: (0, 0)),
            pl.BlockSpec((1, 2), lambda i: (0, 0)),
        ],
        out_specs=[
            pl.BlockSpec((MB, 2), lambda i: (i, 0)),
            pl.BlockSpec((1, 1), lambda i: (0, 0)),
        ],
        out_shape=[
            jax.ShapeDtypeStruct((P, 2), jnp.float32),
            jax.ShapeDtypeStruct((1, 1), jnp.float32),
        ],
        scratch_shapes=[pltpu.VMEM((1, 1), jnp.float32)],
    )(Bu, Bv, targetf, M1a, M1b, b1, M2, b2, M3, b3)


NSUB = 16
ECH = E // NSUB
ZCH = 65664
ACC = ZCH * NSUB


def _sc_scatter_body(ru, cu, vu, rv, cv, vv, zeros_hbm,
                     au_ref, av_ref, csu_ref, csv_ref,
                     acc, rbuf, cbuf, vbuf, linbuf, csbuf):
    cid = lax.axis_index("c")
    sid = lax.axis_index("s")

    def side(r_hbm, c_hbm, v_hbm, a_out, cs_out):
        for j in range(NE):
            pltpu.sync_copy(zeros_hbm,
                            acc.at[pl.ds(sid * ZCH, ZCH)])
            plsc.subcore_barrier()
            base = j * E + sid * ECH
            pltpu.sync_copy(r_hbm.at[pl.ds(base, ECH)], rbuf)
            pltpu.sync_copy(c_hbm.at[pl.ds(base, ECH)], cbuf)
            pltpu.sync_copy(v_hbm.at[pl.ds(base, ECH)], vbuf)

            def body(i, _):
                o = i * 16
                r = rbuf[pl.ds(o, 16)]
                c = cbuf[pl.ds(o, 16)]
                linbuf[pl.ds(o, 16)] = (r << 10) + c
                csbuf[pl.ds(o, 16)] = c + N * N
                return 0

            lax.fori_loop(0, ECH // 16, body, 0)
            pltpu.sync_copy(vbuf, acc.at[linbuf], add=True)
            pltpu.sync_copy(vbuf, acc.at[csbuf], add=True)
            plsc.subcore_barrier()
            pltpu.sync_copy(
                acc.at[pl.ds(sid * (N * N // NSUB), N * N // NSUB)],
                a_out.at[pl.ds(j * N * N + sid * (N * N // NSUB),
                               N * N // NSUB)])

            @pl.when(sid == 0)
            def _():
                pltpu.sync_copy(acc.at[pl.ds(N * N, N)],
                                cs_out.at[pl.ds(j * N, N)])

            plsc.subcore_barrier()

    @pl.when(cid == 0)
    def _():
        side(ru, cu, vu, au_ref, csu_ref)

    @pl.when(cid == 1)
    def _():
        side(rv, cv, vv, av_ref, csv_ref)


@jax.jit
def _sc_scatter(ru, cu, vu, rv, cv, vv):
    zeros = jnp.zeros((ZCH,), jnp.float32)
    mesh = plsc.VectorSubcoreMesh(core_axis_name="c", subcore_axis_name="s")
    f = pl.kernel(
        _sc_scatter_body,
        mesh=mesh,
        out_type=[
            jax.ShapeDtypeStruct((NE * N * N,), jnp.float32),
            jax.ShapeDtypeStruct((NE * N * N,), jnp.float32),
            jax.ShapeDtypeStruct((NE * N,), jnp.float32),
            jax.ShapeDtypeStruct((NE * N,), jnp.float32),
        ],
        scratch_types=[
            pltpu.VMEM_SHARED((ACC,), jnp.float32),
            pltpu.VMEM((ECH,), jnp.int32),
            pltpu.VMEM((ECH,), jnp.int32),
            pltpu.VMEM((ECH,), jnp.float32),
            pltpu.VMEM((ECH,), jnp.int32),
            pltpu.VMEM((ECH,), jnp.int32),
        ],
    )
    return f(ru, cu, vu, rv, cv, vv, zeros)


def _side(A, cs, X, Wg, bg, Wgt1, Wgt2):
    f1 = jax.nn.softmax(Wgt1, axis=1)
    f2 = jax.nn.softmax(Wgt2, axis=1)
    f1t = f1.T
    f2t = f2.T
    Y = _xw(X, Wg)
    dinv, Z = _deg_pass(A, cs, f1t, f2t, Y)
    scale1 = jnp.repeat(f1t, DH, axis=1)[:, None, :]
    scale2 = jnp.repeat(f2t, DH, axis=1)[:, None, :]
    bstack = jnp.tile(bg, (2,))[None, :]
    T1 = _stage(A, Z, scale1, Z, dinv, bstack, epilogue=False)
    Xout = _stage(A, T1, scale2, Z, dinv, bstack, epilogue=True)
    return Xout, f1, f2


def kernel(edge_index_u, edge_value_u, X_u, edge_index_v, edge_value_v, X_v,
           index_list, Wgt1_u, Wgt2_u, Wgt1_v, Wgt2_v, Wg_u, bg_u, Wg_v, bg_v,
           M1, b1, M2, b2, M3, b3):
    ru = edge_index_u[:, 0, :].reshape(-1).astype(jnp.int32)
    cu = edge_index_u[:, 1, :].reshape(-1).astype(jnp.int32)
    rv = edge_index_v[:, 0, :].reshape(-1).astype(jnp.int32)
    cv = edge_index_v[:, 1, :].reshape(-1).astype(jnp.int32)
    Afu, Afv, csu, csv = _sc_scatter(ru, cu, edge_value_u.reshape(-1),
                                     rv, cv, edge_value_v.reshape(-1))
    A_u = Afu.reshape(NE, N, N)
    A_v = Afv.reshape(NE, N, N)
    cs_u = csu.reshape(NE, N)
    cs_v = csv.reshape(NE, N)

    Xu_, f1u, f2u = _side(A_u, cs_u, X_u, Wg_u, bg_u, Wgt1_u, Wgt2_u)
    Xv_, f1v, f2v = _side(A_v, cs_v, X_v, Wg_v, bg_v, Wgt1_v, Wgt2_v)

    u_idx = index_list[:, 0]
    v_idx = index_list[:, 1]
    target = index_list[:, 2]
    targetf = target.astype(jnp.float32)

    Bu = Xu_[u_idx]
    Bv = Xv_[v_idx]

    Bp, loss2 = _mlp(Bu, Bv, targetf[:, None], M1[:DS], M1[DS:], b1[None, :],
                     M2, b2[None, :], M3, b3[None, :])
    loss = loss2.reshape(())
    return (Xu_, Xv_, f1u, f2u, f1v, f2v, loss, Bp, targetf)


# speedup vs baseline: 3.2779x; 1.0523x over previous
import functools

import jax
import jax.numpy as jnp
from jax import lax
from jax.experimental import pallas as pl
from jax.experimental.pallas import tpu as pltpu
from jax.experimental.pallas import tpu_sc as plsc

N = 1024
NE = 3
C = 2
E = 32768
P = 4096
DH = 128
DS = C * DH
RB = 256


def _deg_body(a_ref, cs_ref, f1t_ref, f2t_ref, y_ref, dinv_ref, z_ref,
              racc, vacc):
    rb = pl.program_id(0)
    nrb = pl.num_programs(0)

    @pl.when(rb == 0)
    def _():
        racc[...] = lax.dot_general(cs_ref[...], f1t_ref[...],
                                    (((0,), (0,)), ((), ())),
                                    preferred_element_type=jnp.float32)
        vacc[...] = jnp.zeros_like(vacc)

    rblk = racc[pl.ds(rb * RB, RB), :]
    acc = vacc[...]
    for k in range(NE):
        res = lax.dot_general(a_ref[k], rblk, (((0,), (0,)), ((), ())),
                              preferred_element_type=jnp.float32)
        acc = acc + res * f2t_ref[k:k + 1, :]
    vacc[...] = acc

    @pl.when(rb == nrb - 1)
    def _():
        deg = 1.0 + vacc[...]
        dinv = jnp.where(deg > 0.0, lax.rsqrt(deg), 0.0)
        dinv_ref[...] = dinv
        y = y_ref[...]
        z_ref[...] = jnp.concatenate([dinv[:, 0:1] * y, dinv[:, 1:2] * y],
                                     axis=1)


def _deg_pass(A, cs, f1t, f2t, Y):
    grid = (N // RB,)
    return pl.pallas_call(
        _deg_body,
        grid=grid,
        in_specs=[
            pl.BlockSpec((NE, RB, N), lambda rb: (0, rb, 0)),
            pl.BlockSpec((NE, N), lambda rb: (0, 0)),
            pl.BlockSpec((NE, C), lambda rb: (0, 0)),
            pl.BlockSpec((NE, C), lambda rb: (0, 0)),
            pl.BlockSpec((N, DH), lambda rb: (0, 0)),
        ],
        out_specs=[
            pl.BlockSpec((N, C), lambda rb: (0, 0)),
            pl.BlockSpec((N, DS), lambda rb: (0, 0)),
        ],
        out_shape=[
            jax.ShapeDtypeStruct((N, C), jnp.float32),
            jax.ShapeDtypeStruct((N, DS), jnp.float32),
        ],
        scratch_shapes=[
            pltpu.VMEM((N, C), jnp.float32),
            pltpu.VMEM((N, C), jnp.float32),
        ],
    )(A, cs, f1t, f2t, Y)


def _stage_body(a_ref, zin_ref, scale_ref, z_ref, dinv_ref, b_ref, out_ref,
                acc, *, epilogue):
    j = pl.program_id(1)

    @pl.when(j == 0)
    def _():
        acc[...] = jnp.zeros_like(acc)

    res = lax.dot_general(a_ref[0], zin_ref[...], (((0,), (0,)), ((), ())),
                          preferred_element_type=jnp.float32)
    acc[...] += res * scale_ref[0]

    @pl.when(j == NE - 1)
    def _():
        if epilogue:
            dinvb = dinv_ref[...]
            dcols = jnp.concatenate(
                [jnp.broadcast_to(dinvb[:, 0:1], (RB, DH)),
                 jnp.broadcast_to(dinvb[:, 1:2], (RB, DH))], axis=1)
            out_ref[...] = jnp.maximum(
                dcols * (acc[...] + z_ref[...]) + b_ref[...], 0.0)
        else:
            out_ref[...] = acc[...]


def _stage(A, Zin, scale, Z, dinv, bstack, epilogue):
    grid = (N // RB, NE)
    body = functools.partial(_stage_body, epilogue=epilogue)
    return pl.pallas_call(
        body,
        grid=grid,
        in_specs=[
            pl.BlockSpec((1, N, RB), lambda io, j: (j, 0, io)),
            pl.BlockSpec((N, DS), lambda io, j: (0, 0)),
            pl.BlockSpec((1, 1, DS), lambda io, j: (j, 0, 0)),
            pl.BlockSpec((RB, DS), lambda io, j: (io, 0)),
            pl.BlockSpec((RB, C), lambda io, j: (io, 0)),
            pl.BlockSpec((1, DS), lambda io, j: (0, 0)),
        ],
        out_specs=pl.BlockSpec((RB, DS), lambda io, j: (io, 0)),
        out_shape=jax.ShapeDtypeStruct((N, DS), jnp.float32),
        scratch_shapes=[pltpu.VMEM((RB, DS), jnp.float32)],
    )(A, Zin, scale, Z, dinv, bstack)


def _xw_body(x_ref, w_ref, out_ref):
    out_ref[...] = jnp.dot(x_ref[...], w_ref[...],
                           preferred_element_type=jnp.float32)


def _xw(X, W):
    return pl.pallas_call(
        _xw_body,
        out_shape=jax.ShapeDtypeStruct((N, DH), jnp.float32),
    )(X, W)


MB = 512


def _mlp_body(bu_ref, bv_ref, t_ref, m1a_ref, m1b_ref, b1_ref, m2_ref,
              b2_ref, m3_ref, b3_ref, bp_ref, loss_ref, lacc):
    i = pl.program_id(0)

    @pl.when(i == 0)
    def _():
        lacc[...] = jnp.zeros_like(lacc)

    h = jnp.dot(bu_ref[...], m1a_ref[...], preferred_element_type=jnp.float32)
    h += jnp.dot(bv_ref[...], m1b_ref[...], preferred_element_type=jnp.float32)
    h = jnp.maximum(h + b1_ref[...], 0.0)
    h = jnp.maximum(jnp.dot(h, m2_ref[...], preferred_element_type=jnp.float32)
                    + b2_ref[...], 0.0)
    logits = jnp.dot(h, m3_ref[...], preferred_element_type=jnp.float32) \
        + b3_ref[...]
    m = jnp.max(logits, axis=-1, keepdims=True)
    e = jnp.exp(logits - m)
    bp = e / jnp.sum(e, axis=-1, keepdims=True)
    bp_ref[...] = bp

    mm = jnp.max(bp, axis=-1, keepdims=True)
    lse = mm + jnp.log(jnp.sum(jnp.exp(bp - mm), axis=-1, keepdims=True))
    t = t_ref[...]
    bpt = bp[:, 0:1] * (1.0 - t) + bp[:, 1:2] * t
    lacc[...] += jnp.sum(lse - bpt, axis=0, keepdims=True)

    @pl.when(i == pl.num_programs(0) - 1)
    def _():
        loss_ref[...] = lacc[...] * (1.0 / P)


def _mlp(Bu, Bv, targetf, M1a, M1b, b1, M2, b2, M3, b3):
    grid = (P // MB,)
    return pl.pallas_call(
        _mlp_body,
        grid=grid,
        in_specs=[
            pl.BlockSpec((MB, DS), lambda i: (i, 0)),
            pl.BlockSpec((MB, DS), lambda i: (i, 0)),
            pl.BlockSpec((MB, 1), lambda i: (i, 0)),
            pl.BlockSpec((DS, DS), lambda i: (0, 0)),
            pl.BlockSpec((DS, DS), lambda i: (0, 0)),
            pl.BlockSpec((1, DS), lambda i: (0, 0)),
            pl.BlockSpec((DS, DS // 2), lambda i: (0, 0)),
            pl.BlockSpec((1, DS // 2), lambda i: (0, 0)),
            pl.BlockSpec((DS // 2, 2), lambda i: (0, 0)),
            pl.BlockSpec((1, 2), lambda i: (0, 0)),
        ],
        out_specs=[
            pl.BlockSpec((MB, 2), lambda i: (i, 0)),
            pl.BlockSpec((1, 1), lambda i: (0, 0)),
        ],
        out_shape=[
            jax.ShapeDtypeStruct((P, 2), jnp.float32),
            jax.ShapeDtypeStruct((1, 1), jnp.float32),
        ],
        scratch_shapes=[pltpu.VMEM((1, 1), jnp.float32)],
    )(Bu, Bv, targetf, M1a, M1b, b1, M2, b2, M3, b3)


NSUB = 16
ECH = E // NSUB
ZCH = 65664
ACC = ZCH * NSUB


def _sc_scatter_body(ru, cu, vu, rv, cv, vv, zeros_hbm,
                     au_ref, av_ref, csu_ref, csv_ref,
                     acc, rbuf, cbuf, vbuf, linbuf, csbuf):
    cid = lax.axis_index("c")
    sid = lax.axis_index("s")

    def side(r_hbm, c_hbm, v_hbm, a_out, cs_out):
        for j in range(NE):
            pltpu.sync_copy(zeros_hbm,
                            acc.at[pl.ds(sid * ZCH, ZCH)])
            plsc.subcore_barrier()
            base = j * E + sid * ECH
            pltpu.sync_copy(r_hbm.at[pl.ds(base, ECH)], rbuf)
            pltpu.sync_copy(c_hbm.at[pl.ds(base, ECH)], cbuf)
            pltpu.sync_copy(v_hbm.at[pl.ds(base, ECH)], vbuf)

            def body(i, _):
                o = i * 16
                r = rbuf[pl.ds(o, 16)]
                c = cbuf[pl.ds(o, 16)]
                linbuf[pl.ds(o, 16)] = (r << 10) + c
                csbuf[pl.ds(o, 16)] = c + N * N
                return 0

            lax.fori_loop(0, ECH // 16, body, 0)
            pltpu.sync_copy(vbuf, acc.at[linbuf], add=True)
            pltpu.sync_copy(vbuf, acc.at[csbuf], add=True)
            plsc.subcore_barrier()
            pltpu.sync_copy(
                acc.at[pl.ds(sid * (N * N // NSUB), N * N // NSUB)],
                a_out.at[pl.ds(j * N * N + sid * (N * N // NSUB),
                               N * N // NSUB)])

            @pl.when(sid == 0)
            def _():
                pltpu.sync_copy(acc.at[pl.ds(N * N, N)],
                                cs_out.at[pl.ds(j * N, N)])

            plsc.subcore_barrier()

    @pl.when(cid == 0)
    def _():
        side(ru, cu, vu, au_ref, csu_ref)

    @pl.when(cid == 1)
    def _():
        side(rv, cv, vv, av_ref, csv_ref)


@jax.jit
def _sc_scatter(ru, cu, vu, rv, cv, vv):
    zeros = jnp.zeros((ZCH,), jnp.float32)
    mesh = plsc.VectorSubcoreMesh(core_axis_name="c", subcore_axis_name="s")
    f = pl.kernel(
        _sc_scatter_body,
        mesh=mesh,
        out_type=[
            jax.ShapeDtypeStruct((NE * N * N,), jnp.float32),
            jax.ShapeDtypeStruct((NE * N * N,), jnp.float32),
            jax.ShapeDtypeStruct((NE * N,), jnp.float32),
            jax.ShapeDtypeStruct((NE * N,), jnp.float32),
        ],
        scratch_types=[
            pltpu.VMEM_SHARED((ACC,), jnp.float32),
            pltpu.VMEM((ECH,), jnp.int32),
            pltpu.VMEM((ECH,), jnp.int32),
            pltpu.VMEM((ECH,), jnp.float32),
            pltpu.VMEM((ECH,), jnp.int32),
            pltpu.VMEM((ECH,), jnp.int32),
        ],
    )
    return f(ru, cu, vu, rv, cv, vv, zeros)


GCH = (2 * P) // (2 * NSUB)


def _sc_gather_body(tab, idx, out, ibuf, rows, sem):
    cid = lax.axis_index("c")
    sid = lax.axis_index("s")
    base = (cid * NSUB + sid) * GCH
    pltpu.sync_copy(idx.at[pl.ds(base, GCH)], ibuf)
    pltpu.async_copy(tab.at[ibuf], rows, sem).wait()
    pltpu.sync_copy(rows, out.at[pl.ds(base, GCH)])


@jax.jit
def _sc_gather(table, catidx):
    mesh = plsc.VectorSubcoreMesh(core_axis_name="c", subcore_axis_name="s")
    f = pl.kernel(
        _sc_gather_body,
        mesh=mesh,
        out_type=jax.ShapeDtypeStruct((2 * P, DS), jnp.float32),
        scratch_types=[
            pltpu.VMEM((GCH,), jnp.int32),
            pltpu.VMEM((GCH, DS), jnp.float32),
            pltpu.SemaphoreType.DMA,
        ],
    )
    return f(table, catidx)


def _side(A, cs, X, Wg, bg, Wgt1, Wgt2):
    f1 = jax.nn.softmax(Wgt1, axis=1)
    f2 = jax.nn.softmax(Wgt2, axis=1)
    f1t = f1.T
    f2t = f2.T
    Y = _xw(X, Wg)
    dinv, Z = _deg_pass(A, cs, f1t, f2t, Y)
    scale1 = jnp.repeat(f1t, DH, axis=1)[:, None, :]
    scale2 = jnp.repeat(f2t, DH, axis=1)[:, None, :]
    bstack = jnp.tile(bg, (2,))[None, :]
    T1 = _stage(A, Z, scale1, Z, dinv, bstack, epilogue=False)
    Xout = _stage(A, T1, scale2, Z, dinv, bstack, epilogue=True)
    return Xout, f1, f2


def kernel(edge_index_u, edge_value_u, X_u, edge_index_v, edge_value_v, X_v,
           index_list, Wgt1_u, Wgt2_u, Wgt1_v, Wgt2_v, Wg_u, bg_u, Wg_v, bg_v,
           M1, b1, M2, b2, M3, b3):
    ru = edge_index_u[:, 0, :].reshape(-1).astype(jnp.int32)
    cu = edge_index_u[:, 1, :].reshape(-1).astype(jnp.int32)
    rv = edge_index_v[:, 0, :].reshape(-1).astype(jnp.int32)
    cv = edge_index_v[:, 1, :].reshape(-1).astype(jnp.int32)
    Afu, Afv, csu, csv = _sc_scatter(ru, cu, edge_value_u.reshape(-1),
                                     rv, cv, edge_value_v.reshape(-1))
    A_u = Afu.reshape(NE, N, N)
    A_v = Afv.reshape(NE, N, N)
    cs_u = csu.reshape(NE, N)
    cs_v = csv.reshape(NE, N)

    Xu_, f1u, f2u = _side(A_u, cs_u, X_u, Wg_u, bg_u, Wgt1_u, Wgt2_u)
    Xv_, f1v, f2v = _side(A_v, cs_v, X_v, Wg_v, bg_v, Wgt1_v, Wgt2_v)

    u_idx = index_list[:, 0].astype(jnp.int32)
    v_idx = index_list[:, 1].astype(jnp.int32)
    target = index_list[:, 2]
    targetf = target.astype(jnp.float32)

    table = jnp.concatenate([Xu_, Xv_], axis=0)
    catidx = jnp.concatenate([u_idx, v_idx + N])
    Bcat = _sc_gather(table, catidx)
    Bu = Bcat[:P]
    Bv = Bcat[P:]

    Bp, loss2 = _mlp(Bu, Bv, targetf[:, None], M1[:DS], M1[DS:], b1[None, :],
                     M2, b2[None, :], M3, b3[None, :])
    loss = loss2.reshape(())
    return (Xu_, Xv_, f1u, f2u, f1v, f2v, loss, Bp, targetf)


# speedup vs baseline: 3.4260x; 1.0452x over previous
import jax
import jax.numpy as jnp
from jax import lax
from jax.experimental import pallas as pl
from jax.experimental.pallas import tpu as pltpu
from jax.experimental.pallas import tpu_sc as plsc

N = 1024
NE = 3
C = 2
E = 32768
P = 4096
DH = 128
DS = C * DH
RB = 256

NSUB = 16
ECH = E // NSUB
ZCH = 65664
ACC = ZCH * NSUB
MCH = N * N // NSUB


def _sc_scatter_body(r_hbm, c_hbm, v_hbm, zeros_hbm, a_out, cs_out,
                     acc, rbuf, cbuf, vbuf, linbuf, csbuf):
    cid = lax.axis_index("c")
    sid = lax.axis_index("s")
    ebase = cid * (NE * E)
    abase = cid * (NE * N * N)
    cbase = cid * (NE * N)
    for j in range(NE):
        pltpu.sync_copy(zeros_hbm, acc.at[pl.ds(sid * ZCH, ZCH)])
        base = ebase + j * E + sid * ECH
        pltpu.sync_copy(r_hbm.at[pl.ds(base, ECH)], rbuf)
        pltpu.sync_copy(c_hbm.at[pl.ds(base, ECH)], cbuf)
        pltpu.sync_copy(v_hbm.at[pl.ds(base, ECH)], vbuf)

        def body(i, _):
            o = i * 16
            r = rbuf[pl.ds(o, 16)]
            c = cbuf[pl.ds(o, 16)]
            linbuf[pl.ds(o, 16)] = (r << 10) + c
            csbuf[pl.ds(o, 16)] = c + N * N
            return 0

        lax.fori_loop(0, ECH // 16, body, 0)
        plsc.subcore_barrier()
        pltpu.sync_copy(vbuf, acc.at[linbuf], add=True)
        pltpu.sync_copy(vbuf, acc.at[csbuf], add=True)
        plsc.subcore_barrier()
        pltpu.sync_copy(acc.at[pl.ds(sid * MCH, MCH)],
                        a_out.at[pl.ds(abase + j * N * N + sid * MCH, MCH)])

        @pl.when(sid == 0)
        def _():
            pltpu.sync_copy(acc.at[pl.ds(N * N, N)],
                            cs_out.at[pl.ds(cbase + j * N, N)])

        plsc.subcore_barrier()


@jax.jit
def _sc_scatter(rows, cols, vals):
    zeros = jnp.zeros((ZCH,), jnp.float32)
    mesh = plsc.VectorSubcoreMesh(core_axis_name="c", subcore_axis_name="s")
    f = pl.kernel(
        _sc_scatter_body,
        mesh=mesh,
        out_type=[
            jax.ShapeDtypeStruct((2 * NE * N * N,), jnp.float32),
            jax.ShapeDtypeStruct((2 * NE * N,), jnp.float32),
        ],
        scratch_types=[
            pltpu.VMEM_SHARED((ACC,), jnp.float32),
            pltpu.VMEM((ECH,), jnp.int32),
            pltpu.VMEM((ECH,), jnp.int32),
            pltpu.VMEM((ECH,), jnp.float32),
            pltpu.VMEM((ECH,), jnp.int32),
            pltpu.VMEM((ECH,), jnp.int32),
        ],
    )
    return f(rows, cols, vals, zeros)


GCH = (2 * P) // (2 * NSUB)


def _sc_gather_body(tab, idx, out, ibuf, rows, sem):
    cid = lax.axis_index("c")
    sid = lax.axis_index("s")
    base = (cid * NSUB + sid) * GCH
    pltpu.sync_copy(idx.at[pl.ds(base, GCH)], ibuf)
    pltpu.async_copy(tab.at[ibuf], rows, sem).wait()
    pltpu.sync_copy(rows, out.at[pl.ds(base, GCH)])


@jax.jit
def _sc_gather(table, catidx):
    mesh = plsc.VectorSubcoreMesh(core_axis_name="c", subcore_axis_name="s")
    f = pl.kernel(
        _sc_gather_body,
        mesh=mesh,
        out_type=jax.ShapeDtypeStruct((2 * P, DS), jnp.float32),
        scratch_types=[
            pltpu.VMEM((GCH,), jnp.int32),
            pltpu.VMEM((GCH, DS), jnp.float32),
            pltpu.SemaphoreType.DMA,
        ],
    )
    return f(table, catidx)


def _main_body(a_ref, cs_ref, f1t_ref, f2t_ref, s1_ref, s2_ref, x_ref,
               w_ref, b_ref, out_ref, racc, vblk, dinv, zscr, t1scr, sacc):
    p = pl.program_id(1)
    io = pl.program_id(2)
    j = pl.program_id(3)

    @pl.when((p == 0) & (io == 0) & (j == 0))
    def _():
        racc[...] = lax.dot_general(cs_ref[0], f1t_ref[0],
                                    (((0,), (0,)), ((), ())),
                                    preferred_element_type=jnp.float32)

    ablk = a_ref[0, 0]

    @pl.when(p == 0)
    def _():
        @pl.when(j == 0)
        def _():
            vblk[...] = jnp.zeros_like(vblk)

        res = lax.dot_general(ablk, racc[...], (((0,), (0,)), ((), ())),
                              preferred_element_type=jnp.float32)
        vblk[...] += res * f2t_ref[0, pl.ds(j, 1), :][0]

        @pl.when(j == NE - 1)
        def _():
            deg = 1.0 + vblk[...]
            dinvb = jnp.where(deg > 0.0, lax.rsqrt(deg), 0.0)
            dinv[pl.ds(io * RB, RB), :] = dinvb
            y = jnp.dot(x_ref[0], w_ref[0],
                        preferred_element_type=jnp.float32)
            zscr[pl.ds(io * RB, RB), :] = jnp.concatenate(
                [dinvb[:, 0:1] * y, dinvb[:, 1:2] * y], axis=1)

    @pl.when(p == 1)
    def _():
        @pl.when(j == 0)
        def _():
            sacc[...] = jnp.zeros_like(sacc)

        res = lax.dot_general(ablk, zscr[...], (((0,), (0,)), ((), ())),
                              preferred_element_type=jnp.float32)
        sacc[...] += res * s1_ref[0, 0]

        @pl.when(j == NE - 1)
        def _():
            t1scr[pl.ds(io * RB, RB), :] = sacc[...]

    @pl.when(p == 2)
    def _():
        @pl.when(j == 0)
        def _():
            sacc[...] = jnp.zeros_like(sacc)

        res = lax.dot_general(ablk, t1scr[...], (((0,), (0,)), ((), ())),
                              preferred_element_type=jnp.float32)
        sacc[...] += res * s2_ref[0, 0]

        @pl.when(j == NE - 1)
        def _():
            dinvb = dinv[pl.ds(io * RB, RB), :]
            dcols = jnp.concatenate(
                [jnp.broadcast_to(dinvb[:, 0:1], (RB, DH)),
                 jnp.broadcast_to(dinvb[:, 1:2], (RB, DH))], axis=1)
            zb = zscr[pl.ds(io * RB, RB), :]
            out_ref[0] = jnp.maximum(
                dcols * (sacc[...] + zb) + b_ref[0], 0.0)


def _main(Aall, csall, f1tall, f2tall, s1all, s2all, Xall, Wall, ball):
    grid = (2, 3, N // RB, NE)
    return pl.pallas_call(
        _main_body,
        grid=grid,
        in_specs=[
            pl.BlockSpec((1, 1, N, RB), lambda s, p, io, j: (s, j, 0, io)),
            pl.BlockSpec((1, NE, N), lambda s, p, io, j: (s, 0, 0)),
            pl.BlockSpec((1, NE, C), lambda s, p, io, j: (s, 0, 0)),
            pl.BlockSpec((1, NE, C), lambda s, p, io, j: (s, 0, 0)),
            pl.BlockSpec((1, 1, 1, DS), lambda s, p, io, j: (s, j, 0, 0)),
            pl.BlockSpec((1, 1, 1, DS), lambda s, p, io, j: (s, j, 0, 0)),
            pl.BlockSpec((1, RB, DS), lambda s, p, io, j: (s, io, 0)),
            pl.BlockSpec((1, DS, DH), lambda s, p, io, j: (s, 0, 0)),
            pl.BlockSpec((1, 1, DS), lambda s, p, io, j: (s, 0, 0)),
        ],
        out_specs=pl.BlockSpec((1, RB, DS), lambda s, p, io, j: (s, io, 0)),
        out_shape=jax.ShapeDtypeStruct((2, N, DS), jnp.float32),
        scratch_shapes=[
            pltpu.VMEM((N, C), jnp.float32),
            pltpu.VMEM((RB, C), jnp.float32),
            pltpu.VMEM((N, C), jnp.float32),
            pltpu.VMEM((N, DS), jnp.float32),
            pltpu.VMEM((N, DS), jnp.float32),
            pltpu.VMEM((RB, DS), jnp.float32),
        ],
    )(Aall, csall, f1tall, f2tall, s1all, s2all, Xall, Wall, ball)


MB = 512


def _mlp_body(bu_ref, bv_ref, t_ref, m1a_ref, m1b_ref, b1_ref, m2_ref,
              b2_ref, m3_ref, b3_ref, bp_ref, loss_ref, lacc):
    i = pl.program_id(0)

    @pl.when(i == 0)
    def _():
        lacc[...] = jnp.zeros_like(lacc)

    h = jnp.dot(bu_ref[...], m1a_ref[...], preferred_element_type=jnp.float32)
    h += jnp.dot(bv_ref[...], m1b_ref[...], preferred_element_type=jnp.float32)
    h = jnp.maximum(h + b1_ref[...], 0.0)
    h = jnp.maximum(jnp.dot(h, m2_ref[...], preferred_element_type=jnp.float32)
                    + b2_ref[...], 0.0)
    logits = jnp.dot(h, m3_ref[...], preferred_element_type=jnp.float32) \
        + b3_ref[...]
    m = jnp.max(logits, axis=-1, keepdims=True)
    e = jnp.exp(logits - m)
    bp = e / jnp.sum(e, axis=-1, keepdims=True)
    bp_ref[...] = bp

    mm = jnp.max(bp, axis=-1, keepdims=True)
    lse = mm + jnp.log(jnp.sum(jnp.exp(bp - mm), axis=-1, keepdims=True))
    t = t_ref[...]
    bpt = bp[:, 0:1] * (1.0 - t) + bp[:, 1:2] * t
    lacc[...] += jnp.sum(lse - bpt, axis=0, keepdims=True)

    @pl.when(i == pl.num_programs(0) - 1)
    def _():
        loss_ref[...] = lacc[...] * (1.0 / P)


def _mlp(Bu, Bv, targetf, M1a, M1b, b1, M2, b2, M3, b3):
    grid = (P // MB,)
    return pl.pallas_call(
        _mlp_body,
        grid=grid,
        in_specs=[
            pl.BlockSpec((MB, DS), lambda i: (i, 0)),
            pl.BlockSpec((MB, DS), lambda i: (i, 0)),
            pl.BlockSpec((MB, 1), lambda i: (i, 0)),
            pl.BlockSpec((DS, DS), lambda i: (0, 0)),
            pl.BlockSpec((DS, DS), lambda i: (0, 0)),
            pl.BlockSpec((1, DS), lambda i: (0, 0)),
            pl.BlockSpec((DS, DS // 2), lambda i: (0, 0)),
            pl.BlockSpec((1, DS // 2), lambda i: (0, 0)),
            pl.BlockSpec((DS // 2, 2), lambda i: (0, 0)),
            pl.BlockSpec((1, 2), lambda i: (0, 0)),
        ],
        out_specs=[
            pl.BlockSpec((MB, 2), lambda i: (i, 0)),
            pl.BlockSpec((1, 1), lambda i: (0, 0)),
        ],
        out_shape=[
            jax.ShapeDtypeStruct((P, 2), jnp.float32),
            jax.ShapeDtypeStruct((1, 1), jnp.float32),
        ],
        scratch_shapes=[pltpu.VMEM((1, 1), jnp.float32)],
    )(Bu, Bv, targetf, M1a, M1b, b1, M2, b2, M3, b3)


def kernel(edge_index_u, edge_value_u, X_u, edge_index_v, edge_value_v, X_v,
           index_list, Wgt1_u, Wgt2_u, Wgt1_v, Wgt2_v, Wg_u, bg_u, Wg_v, bg_v,
           M1, b1, M2, b2, M3, b3):
    rows = jnp.concatenate([edge_index_u[:, 0, :].reshape(-1),
                            edge_index_v[:, 0, :].reshape(-1)]) \
        .astype(jnp.int32)
    cols = jnp.concatenate([edge_index_u[:, 1, :].reshape(-1),
                            edge_index_v[:, 1, :].reshape(-1)]) \
        .astype(jnp.int32)
    vals = jnp.concatenate([edge_value_u.reshape(-1),
                            edge_value_v.reshape(-1)])
    Afall, csfall = _sc_scatter(rows, cols, vals)
    Aall = Afall.reshape(2, NE, N, N)
    csall = csfall.reshape(2, NE, N)

    f1u = jax.nn.softmax(Wgt1_u, axis=1)
    f2u = jax.nn.softmax(Wgt2_u, axis=1)
    f1v = jax.nn.softmax(Wgt1_v, axis=1)
    f2v = jax.nn.softmax(Wgt2_v, axis=1)
    f1tall = jnp.stack([f1u.T, f1v.T])
    f2tall = jnp.stack([f2u.T, f2v.T])
    s1all = jnp.repeat(f1tall, DH, axis=2)[:, :, None, :]
    s2all = jnp.repeat(f2tall, DH, axis=2)[:, :, None, :]
    Xall = jnp.stack([X_u, X_v])
    Wall = jnp.stack([Wg_u, Wg_v])
    ball = jnp.stack([jnp.tile(bg_u, (2,))[None, :],
                      jnp.tile(bg_v, (2,))[None, :]])

    Xout = _main(Aall, csall, f1tall, f2tall, s1all, s2all, Xall, Wall, ball)
    Xu_ = Xout[0]
    Xv_ = Xout[1]

    u_idx = index_list[:, 0].astype(jnp.int32)
    v_idx = index_list[:, 1].astype(jnp.int32)
    target = index_list[:, 2]
    targetf = target.astype(jnp.float32)

    table = Xout.reshape(2 * N, DS)
    catidx = jnp.concatenate([u_idx, v_idx + N])
    Bcat = _sc_gather(table, catidx)
    Bu = Bcat[:P]
    Bv = Bcat[P:]

    Bp, loss2 = _mlp(Bu, Bv, targetf[:, None], M1[:DS], M1[DS:], b1[None, :],
                     M2, b2[None, :], M3, b3[None, :])
    loss = loss2.reshape(())
    return (Xu_, Xv_, f1u, f2u, f1v, f2v, loss, Bp, targetf)
